# baseline (speedup 1.0000x reference)
import functools

import jax
import jax.numpy as jnp
from jax import lax
from jax.experimental import pallas as pl
from jax.experimental.pallas import tpu as pltpu
from jax.experimental.pallas import tpu_sc as plsc

N_IDS = 425984
NUM_BUCKETS = 1000000
NC = 2
NS = 16
NW = NC * NS
B_PER_W = N_IDS // NW
SEG = 62464
SEG_LAST = 999936 - (NS - 1) * SEG
TAIL_OFF = 999936
SH_SIZE = TAIL_OFF + 512

_mesh = plsc.VectorSubcoreMesh(core_axis_name="c", subcore_axis_name="s")


@functools.partial(
    pl.kernel,
    mesh=_mesh,
    out_type=jax.ShapeDtypeStruct((N_IDS,), jnp.float32),
    scratch_types=[
        pltpu.VMEM((B_PER_W,), jnp.int32),
        pltpu.VMEM((B_PER_W,), jnp.float32),
        pltpu.VMEM_SHARED((SH_SIZE,), jnp.float32),
        pltpu.SemaphoreType.DMA,
        pltpu.SemaphoreType.DMA,
    ],
)
def _gather_kernel(ids_hbm, table_hbm, out_hbm, idx_v, rows_v,
                   tbl_sh, g_sem, t_sem):
    cid = lax.axis_index("c")
    sid = lax.axis_index("s")
    wid = sid * NC + cid
    base = wid * B_PER_W

    @pl.when(sid < NS - 1)
    def _stage_main():
        pltpu.async_copy(table_hbm.at[pl.ds(sid * SEG, SEG)],
                         tbl_sh.at[pl.ds(sid * SEG, SEG)], t_sem)

    @pl.when(sid == NS - 1)
    def _stage_last():
        pltpu.async_copy(table_hbm.at[pl.ds((NS - 1) * SEG, SEG_LAST)],
                         tbl_sh.at[pl.ds((NS - 1) * SEG, SEG_LAST)], t_sem)

    @pl.when(sid == 0)
    def _stage_tail():
        pltpu.sync_copy(table_hbm.at[pl.ds(TAIL_OFF, 64)],
                        rows_v.at[pl.ds(0, 64)])
        pltpu.sync_copy(rows_v.at[pl.ds(0, 512)],
                        tbl_sh.at[pl.ds(TAIL_OFF, 512)])

    pltpu.sync_copy(ids_hbm.at[pl.ds(base, B_PER_W)], idx_v)

    @pl.when(sid < NS - 1)
    def _wait_main():
        pltpu.make_async_copy(table_hbm.at[pl.ds(sid * SEG, SEG)],
                              tbl_sh.at[pl.ds(sid * SEG, SEG)], t_sem).wait()

    @pl.when(sid == NS - 1)
    def _wait_last():
        pltpu.make_async_copy(
            table_hbm.at[pl.ds((NS - 1) * SEG, SEG_LAST)],
            tbl_sh.at[pl.ds((NS - 1) * SEG, SEG_LAST)], t_sem).wait()

    plsc.subcore_barrier()
    H = B_PER_W // 2
    c1 = pltpu.async_copy(tbl_sh.at[idx_v.at[pl.ds(0, H)]],
                          rows_v.at[pl.ds(0, H)], g_sem)
    c2 = pltpu.async_copy(tbl_sh.at[idx_v.at[pl.ds(H, H)]],
                          rows_v.at[pl.ds(H, H)], g_sem)
    c1.wait()
    c2.wait()
    pltpu.sync_copy(rows_v, out_hbm.at[pl.ds(base, B_PER_W)])


def kernel(ids, kernel):
    gathered = _gather_kernel(ids.astype(jnp.int32), kernel)
    return gathered.reshape(N_IDS, 1)


# speedup vs baseline: 1.0080x; 1.0080x over previous
import functools

import jax
import jax.numpy as jnp
from jax import lax
from jax.experimental import pallas as pl
from jax.experimental.pallas import tpu as pltpu
from jax.experimental.pallas import tpu_sc as plsc

N_IDS = 425984
NUM_BUCKETS = 1000000
NC = 2
NS = 16
NW = NC * NS
B_PER_W = N_IDS // NW
SEG = 62464
SEG_LAST = 999936 - (NS - 1) * SEG
TAIL_OFF = 999936
SH_SIZE = TAIL_OFF + 512

_mesh = plsc.VectorSubcoreMesh(core_axis_name="c", subcore_axis_name="s")


@functools.partial(
    pl.kernel,
    mesh=_mesh,
    out_type=jax.ShapeDtypeStruct((N_IDS,), jnp.float32),
    scratch_types=[
        pltpu.VMEM((B_PER_W,), jnp.int32),
        pltpu.VMEM((B_PER_W,), jnp.float32),
        pltpu.VMEM_SHARED((SH_SIZE,), jnp.float32),
        pltpu.SemaphoreType.DMA,
        pltpu.SemaphoreType.DMA,
    ],
)
def _gather_kernel(ids_hbm, table_hbm, out_hbm, idx_v, rows_v,
                   tbl_sh, g_sem, t_sem):
    cid = lax.axis_index("c")
    sid = lax.axis_index("s")
    wid = sid * NC + cid
    base = wid * B_PER_W

    @pl.when(sid < NS - 1)
    def _stage_main():
        pltpu.async_copy(table_hbm.at[pl.ds(sid * SEG, SEG)],
                         tbl_sh.at[pl.ds(sid * SEG, SEG)], t_sem)

    @pl.when(sid == NS - 1)
    def _stage_last():
        pltpu.async_copy(table_hbm.at[pl.ds((NS - 1) * SEG, SEG_LAST)],
                         tbl_sh.at[pl.ds((NS - 1) * SEG, SEG_LAST)], t_sem)

    @pl.when(sid == 0)
    def _stage_tail():
        pltpu.sync_copy(table_hbm.at[pl.ds(TAIL_OFF, 64)],
                        rows_v.at[pl.ds(0, 64)])
        pltpu.sync_copy(rows_v.at[pl.ds(0, 512)],
                        tbl_sh.at[pl.ds(TAIL_OFF, 512)])

    pltpu.sync_copy(ids_hbm.at[pl.ds(base, B_PER_W)], idx_v)

    @pl.when(sid < NS - 1)
    def _wait_main():
        pltpu.make_async_copy(table_hbm.at[pl.ds(sid * SEG, SEG)],
                              tbl_sh.at[pl.ds(sid * SEG, SEG)], t_sem).wait()

    @pl.when(sid == NS - 1)
    def _wait_last():
        pltpu.make_async_copy(
            table_hbm.at[pl.ds((NS - 1) * SEG, SEG_LAST)],
            tbl_sh.at[pl.ds((NS - 1) * SEG, SEG_LAST)], t_sem).wait()

    plsc.subcore_barrier()
    pltpu.async_copy(tbl_sh.at[idx_v], rows_v, g_sem).wait()
    pltpu.sync_copy(rows_v, out_hbm.at[pl.ds(base, B_PER_W)])


def kernel(ids, kernel):
    gathered = _gather_kernel(ids.astype(jnp.int32), kernel)
    return gathered.reshape(N_IDS, 1)


# speedup vs baseline: 1.0099x; 1.0019x over previous
import functools

import jax
import jax.numpy as jnp
from jax import lax
from jax.experimental import pallas as pl
from jax.experimental.pallas import tpu as pltpu
from jax.experimental.pallas import tpu_sc as plsc

N_IDS = 425984
NUM_BUCKETS = 1000000
NC = 2
NS = 16
NW = NC * NS
B_PER_W = N_IDS // NW
SEG = 63488
SEG0 = 999936 - (NS - 1) * SEG
TAIL_OFF = 999936
SH_SIZE = TAIL_OFF + 512

_mesh = plsc.VectorSubcoreMesh(core_axis_name="c", subcore_axis_name="s")


@functools.partial(
    pl.kernel,
    mesh=_mesh,
    out_type=jax.ShapeDtypeStruct((N_IDS,), jnp.float32),
    scratch_types=[
        pltpu.VMEM((B_PER_W,), jnp.int32),
        pltpu.VMEM((B_PER_W,), jnp.float32),
        pltpu.VMEM_SHARED((SH_SIZE,), jnp.float32),
        pltpu.SemaphoreType.DMA,
        pltpu.SemaphoreType.DMA,
    ],
)
def _gather_kernel(ids_hbm, table_hbm, out_hbm, idx_v, rows_v,
                   tbl_sh, g_sem, t_sem):
    cid = lax.axis_index("c")
    sid = lax.axis_index("s")
    wid = sid * NC + cid
    base = wid * B_PER_W
    seg_off = SEG0 + (sid - 1) * SEG

    @pl.when(sid == 0)
    def _stage_t0():
        pltpu.async_copy(table_hbm.at[pl.ds(0, SEG0)],
                         tbl_sh.at[pl.ds(0, SEG0)], t_sem)
        pltpu.sync_copy(table_hbm.at[pl.ds(TAIL_OFF, 64)],
                        rows_v.at[pl.ds(0, 64)])
        pltpu.sync_copy(rows_v.at[pl.ds(0, 512)],
                        tbl_sh.at[pl.ds(TAIL_OFF, 512)])

    @pl.when(sid > 0)
    def _stage_main():
        pltpu.async_copy(table_hbm.at[pl.ds(seg_off, SEG)],
                         tbl_sh.at[pl.ds(seg_off, SEG)], t_sem)

    pltpu.sync_copy(ids_hbm.at[pl.ds(base, B_PER_W)], idx_v)

    @pl.when(sid == 0)
    def _wait_t0():
        pltpu.make_async_copy(table_hbm.at[pl.ds(0, SEG0)],
                              tbl_sh.at[pl.ds(0, SEG0)], t_sem).wait()

    @pl.when(sid > 0)
    def _wait_main():
        pltpu.make_async_copy(table_hbm.at[pl.ds(seg_off, SEG)],
                              tbl_sh.at[pl.ds(seg_off, SEG)], t_sem).wait()

    plsc.subcore_barrier()
    pltpu.async_copy(tbl_sh.at[idx_v], rows_v, g_sem).wait()
    pltpu.sync_copy(rows_v, out_hbm.at[pl.ds(base, B_PER_W)])


def kernel(ids, kernel):
    gathered = _gather_kernel(ids.astype(jnp.int32), kernel)
    return gathered.reshape(N_IDS, 1)
